# Initial kernel scaffold; baseline (speedup 1.0000x reference)
import jax
import jax.numpy as jnp
from jax import lax
from jax.experimental import pallas as pl
from jax.experimental.pallas import tpu as pltpu
from jax.experimental.pallas import tpu_sc as plsc

N = 10000
E = 320000
D = 128
N_CLS = 121

NC = 2
NS = 16
NW = NC * NS
EPT = E // NW
CH = 80
NCHUNK = EPT // CH
RPT = N // NS
ZR = 125

_sc_mesh = plsc.VectorSubcoreMesh(core_axis_name="c", subcore_axis_name="s")


def _sc_segsum(y, edges, with_deg):
    out_type = [jax.ShapeDtypeStruct((NC, N, D), jnp.float32)]
    scratch = [
        pltpu.VMEM((NCHUNK, CH), jnp.int32),
        pltpu.VMEM((NCHUNK, CH), jnp.int32),
        pltpu.VMEM((CH, D), jnp.float32),
        pltpu.VMEM((ZR, D), jnp.float32),
        pltpu.VMEM_SHARED((N, D), jnp.float32),
    ]
    if with_deg:
        out_type.append(jax.ShapeDtypeStruct((NC, N, 16), jnp.float32))
        scratch += [
            pltpu.VMEM((CH, 16), jnp.float32),
            pltpu.VMEM((ZR, 16), jnp.float32),
            pltpu.VMEM_SHARED((N, 16), jnp.float32),
        ]

    def body(y_hbm, e_hbm, *refs):
        if with_deg:
            (out_hbm, deg_hbm, src_v, dst_v, rows_v, zbuf, acc,
             ones_v, dzero, dacc) = refs
        else:
            out_hbm, src_v, dst_v, rows_v, zbuf, acc = refs
        c = lax.axis_index("c")
        s = lax.axis_index("s")
        wid = c * NS + s
        row0 = s * RPT

        @pl.loop(0, ZR)
        def _(r):
            @pl.loop(0, D, step=16)
            def _(col):
                zbuf[r, pl.ds(col, 16)] = jnp.zeros((16,), jnp.float32)

        @pl.loop(0, RPT, step=ZR)
        def _(r0):
            pltpu.sync_copy(zbuf, acc.at[pl.ds(row0 + r0, ZR)])

        if with_deg:
            @pl.loop(0, ZR)
            def _(r):
                dzero[r, pl.ds(0, 16)] = jnp.zeros((16,), jnp.float32)

            @pl.loop(0, CH)
            def _(r):
                ones_v[r, pl.ds(0, 16)] = jnp.ones((16,), jnp.float32)

            @pl.loop(0, RPT, step=ZR)
            def _(r0):
                pltpu.sync_copy(dzero, dacc.at[pl.ds(row0 + r0, ZR)])

        pltpu.sync_copy(e_hbm.at[0, wid], src_v)
        pltpu.sync_copy(e_hbm.at[1, wid], dst_v)
        plsc.subcore_barrier()

        @pl.loop(0, NCHUNK)
        def _(j):
            pltpu.sync_copy(y_hbm.at[src_v.at[j]], rows_v)
            pltpu.sync_copy(rows_v, acc.at[dst_v.at[j]], add=True)
            if with_deg:
                pltpu.sync_copy(ones_v, dacc.at[dst_v.at[j]], add=True)

        plsc.subcore_barrier()
        pltpu.sync_copy(acc.at[pl.ds(row0, RPT)],
                        out_hbm.at[c, pl.ds(row0, RPT)])
        if with_deg:
            pltpu.sync_copy(dacc.at[pl.ds(row0, RPT)],
                            deg_hbm.at[c, pl.ds(row0, RPT)])

    kern = pl.kernel(body, out_type=out_type, mesh=_sc_mesh,
                     scratch_types=scratch)
    return kern(y, edges)


_BLK = 1000


def _tc_in_body(x_ref, wl_ref, wc_ref, bc_ref, y_ref, z_ref):
    xb = x_ref[...]
    y_ref[...] = jnp.dot(xb, wl_ref[...], preferred_element_type=jnp.float32)
    z_ref[...] = (jnp.dot(xb, wc_ref[...], preferred_element_type=jnp.float32)
                  + bc_ref[...])


def _tc_mid_body(a0_ref, a1_ref, d0_ref, d1_ref, z1_ref, wl_ref, wc_ref,
                 bc_ref, y2_ref, z2_ref):
    deg = jnp.clip(d0_ref[...][:, :1] + d1_ref[...][:, :1], 1.0, None)
    h = (a0_ref[...] + a1_ref[...]) / deg + z1_ref[...]
    h = jnp.where(h > 0, h, jnp.expm1(h))
    y2_ref[...] = jnp.dot(h, wl_ref[...], preferred_element_type=jnp.float32)
    z2_ref[...] = (jnp.dot(h, wc_ref[...], preferred_element_type=jnp.float32)
                   + bc_ref[...])


def _tc_out_body(a0_ref, a1_ref, d0_ref, d1_ref, z2_ref, o_ref):
    deg = jnp.clip(d0_ref[...][:, :1] + d1_ref[...][:, :1], 1.0, None)
    o_ref[...] = (a0_ref[...] + a1_ref[...]) / deg + z2_ref[...]


def _row_spec(width):
    return pl.BlockSpec((_BLK, width), lambda i: (i, 0))


def _full_spec(shape):
    return pl.BlockSpec(shape, lambda i: (0,) * len(shape))


def kernel(x, edge_index, Wl1, bl1, Wr1, Wlin1, blin1, Wl2, bl2, Wr2,
           Wlin2, blin2):
    W1c = Wr1 + Wlin1
    b1c = (bl1 + blin1).reshape(1, D)
    pad = ((0, 0), (0, D - N_CLS))
    Wl2p = jnp.pad(Wl2, pad)
    W2c = jnp.pad(Wr2 + Wlin2, pad)
    b2c = jnp.pad((bl2 + blin2).reshape(1, N_CLS), ((0, 0), (0, D - N_CLS)))
    edges = edge_index.reshape(2, NW, NCHUNK, CH)

    grid = (N // _BLK,)
    y1, z1 = pl.pallas_call(
        _tc_in_body,
        grid=grid,
        in_specs=[_row_spec(D), _full_spec((D, D)), _full_spec((D, D)),
                  _full_spec((1, D))],
        out_specs=[_row_spec(D), _row_spec(D)],
        out_shape=[jax.ShapeDtypeStruct((N, D), jnp.float32)] * 2,
    )(x, Wl1, W1c, b1c)

    agg1, degp = _sc_segsum(y1, edges, with_deg=True)

    y2, z2 = pl.pallas_call(
        _tc_mid_body,
        grid=grid,
        in_specs=[_row_spec(D), _row_spec(D), _row_spec(16), _row_spec(16),
                  _row_spec(D), _full_spec((D, D)), _full_spec((D, D)),
                  _full_spec((1, D))],
        out_specs=[_row_spec(D), _row_spec(D)],
        out_shape=[jax.ShapeDtypeStruct((N, D), jnp.float32)] * 2,
    )(agg1[0], agg1[1], degp[0], degp[1], z1, Wl2p, W2c, b2c)

    agg2 = _sc_segsum(y2, edges, with_deg=False)[0]

    out = pl.pallas_call(
        _tc_out_body,
        grid=grid,
        in_specs=[_row_spec(D), _row_spec(D), _row_spec(16), _row_spec(16),
                  _row_spec(D)],
        out_specs=_row_spec(D),
        out_shape=jax.ShapeDtypeStruct((N, D), jnp.float32),
    )(agg2[0], agg2[1], degp[0], degp[1], z2)

    return out[:, :N_CLS]


# speedup vs baseline: 3.2858x; 3.2858x over previous
import jax
import jax.numpy as jnp
from jax import lax
from jax.experimental import pallas as pl
from jax.experimental.pallas import tpu as pltpu
from jax.experimental.pallas import tpu_sc as plsc

N = 10000
E = 320000
D = 128
N_CLS = 121

NC = 2
NS = 16
NW = NC * NS
EPT = E // NW
CH = 128
EPTP = 10240
NCHUNK = EPTP // CH
NP = 10240
RPT = NP // NS
ZR = 16

_sc_mesh = plsc.VectorSubcoreMesh(core_axis_name="c", subcore_axis_name="s")


def _sc_segsum(y, edges):
    def body(y_hbm, e_hbm, out_hbm, src_v, dst_v, rows_v, zbuf, acc):
        c = lax.axis_index("c")
        s = lax.axis_index("s")
        wid = c * NS + s
        row0 = s * RPT

        @pl.loop(0, ZR)
        def _(r):
            @pl.loop(0, D, step=16)
            def _(col):
                zbuf[r, pl.ds(col, 16)] = jnp.zeros((16,), jnp.float32)

        @pl.loop(0, RPT, step=ZR)
        def _(r0):
            pltpu.sync_copy(zbuf, acc.at[pl.ds(row0 + r0, ZR)])

        pltpu.sync_copy(e_hbm.at[0, wid], src_v)
        pltpu.sync_copy(e_hbm.at[1, wid], dst_v)
        plsc.subcore_barrier()

        @pl.loop(0, NCHUNK)
        def _(j):
            pltpu.sync_copy(y_hbm.at[src_v.at[j]], rows_v)
            pltpu.sync_copy(rows_v, acc.at[dst_v.at[j]], add=True)

        plsc.subcore_barrier()
        pltpu.sync_copy(acc.at[pl.ds(row0, RPT)],
                        out_hbm.at[c, pl.ds(row0, RPT)])

    kern = pl.kernel(
        body,
        out_type=jax.ShapeDtypeStruct((NC, NP, D), jnp.float32),
        mesh=_sc_mesh,
        scratch_types=[
            pltpu.VMEM((NCHUNK, CH), jnp.int32),
            pltpu.VMEM((NCHUNK, CH), jnp.int32),
            pltpu.VMEM((CH, D), jnp.float32),
            pltpu.VMEM((ZR, D), jnp.float32),
            pltpu.VMEM_SHARED((NP, D), jnp.float32),
        ],
    )
    return kern(y, edges)


def _sc_degree(edges):
    def body(e_hbm, deg_hbm, dst_v, ones_v, dzero, dacc):
        c = lax.axis_index("c")
        s = lax.axis_index("s")
        wid = c * NS + s
        row0 = s * RPT

        @pl.loop(0, CH)
        def _(r):
            @pl.loop(0, D, step=16)
            def _(col):
                ones_v[r, pl.ds(col, 16)] = jnp.ones((16,), jnp.float32)

        @pl.loop(0, ZR)
        def _(r):
            @pl.loop(0, D, step=16)
            def _(col):
                dzero[r, pl.ds(col, 16)] = jnp.zeros((16,), jnp.float32)

        @pl.loop(0, RPT, step=ZR)
        def _(r0):
            pltpu.sync_copy(dzero, dacc.at[pl.ds(row0 + r0, ZR)])

        pltpu.sync_copy(e_hbm.at[1, wid], dst_v)
        plsc.subcore_barrier()

        @pl.loop(0, NCHUNK)
        def _(j):
            pltpu.sync_copy(ones_v, dacc.at[dst_v.at[j]], add=True)

        plsc.subcore_barrier()
        pltpu.sync_copy(dacc.at[pl.ds(row0, RPT)],
                        deg_hbm.at[c, pl.ds(row0, RPT)])

    kern = pl.kernel(
        body,
        out_type=jax.ShapeDtypeStruct((NC, NP, D), jnp.float32),
        mesh=_sc_mesh,
        scratch_types=[
            pltpu.VMEM((NCHUNK, CH), jnp.int32),
            pltpu.VMEM((CH, D), jnp.float32),
            pltpu.VMEM((ZR, D), jnp.float32),
            pltpu.VMEM_SHARED((NP, D), jnp.float32),
        ],
    )
    return kern(edges)


_BLK = 1000


def _tc_in_body(x_ref, wl_ref, wc_ref, bc_ref, y_ref, z_ref):
    xb = x_ref[...]
    y_ref[...] = jnp.dot(xb, wl_ref[...], preferred_element_type=jnp.float32)
    z_ref[...] = (jnp.dot(xb, wc_ref[...], preferred_element_type=jnp.float32)
                  + bc_ref[...])


def _tc_mid_body(a0_ref, a1_ref, d0_ref, d1_ref, z1_ref, wl_ref, wc_ref,
                 bc_ref, y2_ref, z2_ref):
    deg = jnp.clip(d0_ref[...][:, :1] + d1_ref[...][:, :1], 1.0, None)
    h = (a0_ref[...] + a1_ref[...]) / deg + z1_ref[...]
    h = jnp.where(h > 0, h, jnp.exp(jnp.minimum(h, 0.0)) - 1.0)
    y2_ref[...] = jnp.dot(h, wl_ref[...], preferred_element_type=jnp.float32)
    z2_ref[...] = (jnp.dot(h, wc_ref[...], preferred_element_type=jnp.float32)
                   + bc_ref[...])


def _tc_out_body(a0_ref, a1_ref, d0_ref, d1_ref, z2_ref, o_ref):
    deg = jnp.clip(d0_ref[...][:, :1] + d1_ref[...][:, :1], 1.0, None)
    o_ref[...] = (a0_ref[...] + a1_ref[...]) / deg + z2_ref[...]


def _row_spec(width):
    return pl.BlockSpec((_BLK, width), lambda i: (i, 0))


def _full_spec(shape):
    return pl.BlockSpec(shape, lambda i: (0,) * len(shape))


def kernel(x, edge_index, Wl1, bl1, Wr1, Wlin1, blin1, Wl2, bl2, Wr2,
           Wlin2, blin2):
    W1c = Wr1 + Wlin1
    b1c = (bl1 + blin1).reshape(1, D)
    pad = ((0, 0), (0, D - N_CLS))
    Wl2p = jnp.pad(Wl2, pad)
    W2c = jnp.pad(Wr2 + Wlin2, pad)
    b2c = jnp.pad((bl2 + blin2).reshape(1, N_CLS), ((0, 0), (0, D - N_CLS)))
    e = edge_index.reshape(2, NW, EPT)
    epad = jnp.broadcast_to(
        jnp.array([[0], [NP - 1]], dtype=jnp.int32).reshape(2, 1, 1),
        (2, NW, EPTP - EPT))
    edges = jnp.concatenate([e, epad], axis=2).reshape(2, NW, NCHUNK, CH)
    degp = _sc_degree(edges)

    grid = (N // _BLK,)
    y1, z1 = pl.pallas_call(
        _tc_in_body,
        grid=grid,
        in_specs=[_row_spec(D), _full_spec((D, D)), _full_spec((D, D)),
                  _full_spec((1, D))],
        out_specs=[_row_spec(D), _row_spec(D)],
        out_shape=[jax.ShapeDtypeStruct((N, D), jnp.float32)] * 2,
    )(x, Wl1, W1c, b1c)

    agg1 = _sc_segsum(y1, edges)

    y2, z2 = pl.pallas_call(
        _tc_mid_body,
        grid=grid,
        in_specs=[_row_spec(D), _row_spec(D), _row_spec(D), _row_spec(D),
                  _row_spec(D), _full_spec((D, D)), _full_spec((D, D)),
                  _full_spec((1, D))],
        out_specs=[_row_spec(D), _row_spec(D)],
        out_shape=[jax.ShapeDtypeStruct((N, D), jnp.float32)] * 2,
    )(agg1[0], agg1[1], degp[0], degp[1], z1, Wl2p, W2c, b2c)

    agg2 = _sc_segsum(y2, edges)

    out = pl.pallas_call(
        _tc_out_body,
        grid=grid,
        in_specs=[_row_spec(D), _row_spec(D), _row_spec(D), _row_spec(D),
                  _row_spec(D)],
        out_specs=_row_spec(D),
        out_shape=jax.ShapeDtypeStruct((N, D), jnp.float32),
    )(agg2[0], agg2[1], degp[0], degp[1], z2)

    return out[:, :N_CLS]


# speedup vs baseline: 3.7216x; 1.1326x over previous
import jax
import jax.numpy as jnp
from jax import lax
from jax.experimental import pallas as pl
from jax.experimental.pallas import tpu as pltpu
from jax.experimental.pallas import tpu_sc as plsc

N = 10000
E = 320000
D = 128
N_CLS = 121

NC = 2
NS = 16
NW = NC * NS
EPT = E // NW
CH = 128
EPTP = 10240
NCHUNK = EPTP // CH
IB = 40
NB = NCHUNK // IB
NP = 10240
RPT = NP // NS
ZR = 16

_sc_mesh = plsc.VectorSubcoreMesh(core_axis_name="c", subcore_axis_name="s")


def _sc_segsum(y, edges):
    def body(y_hbm, e_hbm, out_hbm, src_v, dst_v, rows0, rows1, zbuf, acc,
             sem0, sem1):
        c = lax.axis_index("c")
        s = lax.axis_index("s")
        wid = c * NS + s
        row0 = s * RPT

        @pl.loop(0, ZR)
        def _(r):
            @pl.loop(0, D, step=16)
            def _(col):
                zbuf[r, pl.ds(col, 16)] = jnp.zeros((16,), jnp.float32)

        @pl.loop(0, RPT, step=ZR)
        def _(r0):
            pltpu.sync_copy(zbuf, acc.at[pl.ds(row0 + r0, ZR)])

        plsc.subcore_barrier()

        @pl.loop(0, NB)
        def _(b):
            pltpu.sync_copy(e_hbm.at[0, wid, pl.ds(b * IB, IB)], src_v)
            pltpu.sync_copy(e_hbm.at[1, wid, pl.ds(b * IB, IB)], dst_v)
            pltpu.async_copy(y_hbm.at[src_v.at[0]], rows0, sem0)

            @pl.loop(0, IB // 2)
            def _(i):
                j0 = 2 * i
                j1 = j0 + 1
                pltpu.async_copy(y_hbm.at[src_v.at[j1]], rows1, sem1)
                pltpu.make_async_copy(y_hbm.at[src_v.at[j0]], rows0,
                                      sem0).wait()
                pltpu.sync_copy(rows0, acc.at[dst_v.at[j0]], add=True)

                @pl.when(j0 + 2 < IB)
                def _():
                    pltpu.async_copy(y_hbm.at[src_v.at[j0 + 2]], rows0, sem0)

                pltpu.make_async_copy(y_hbm.at[src_v.at[j1]], rows1,
                                      sem1).wait()
                pltpu.sync_copy(rows1, acc.at[dst_v.at[j1]], add=True)

        plsc.subcore_barrier()
        pltpu.sync_copy(acc.at[pl.ds(row0, RPT)],
                        out_hbm.at[c, pl.ds(row0, RPT)])

    kern = pl.kernel(
        body,
        out_type=jax.ShapeDtypeStruct((NC, NP, D), jnp.float32),
        mesh=_sc_mesh,
        scratch_types=[
            pltpu.VMEM((IB, CH), jnp.int32),
            pltpu.VMEM((IB, CH), jnp.int32),
            pltpu.VMEM((CH, D), jnp.float32),
            pltpu.VMEM((CH, D), jnp.float32),
            pltpu.VMEM((ZR, D), jnp.float32),
            pltpu.VMEM_SHARED((NP, D), jnp.float32),
            pltpu.SemaphoreType.DMA,
            pltpu.SemaphoreType.DMA,
        ],
    )
    return kern(y, edges)


def _sc_degree(edges):
    def body(e_hbm, deg_hbm, dst_v, ones_v, dzero, dacc):
        c = lax.axis_index("c")
        s = lax.axis_index("s")
        wid = c * NS + s
        row0 = s * RPT

        @pl.loop(0, CH)
        def _(r):
            @pl.loop(0, D, step=16)
            def _(col):
                ones_v[r, pl.ds(col, 16)] = jnp.ones((16,), jnp.float32)

        @pl.loop(0, ZR)
        def _(r):
            @pl.loop(0, D, step=16)
            def _(col):
                dzero[r, pl.ds(col, 16)] = jnp.zeros((16,), jnp.float32)

        @pl.loop(0, RPT, step=ZR)
        def _(r0):
            pltpu.sync_copy(dzero, dacc.at[pl.ds(row0 + r0, ZR)])

        pltpu.sync_copy(e_hbm.at[1, wid], dst_v)
        plsc.subcore_barrier()

        @pl.loop(0, NCHUNK)
        def _(j):
            pltpu.sync_copy(ones_v, dacc.at[dst_v.at[j]], add=True)

        plsc.subcore_barrier()
        pltpu.sync_copy(dacc.at[pl.ds(row0, RPT)],
                        deg_hbm.at[c, pl.ds(row0, RPT)])

    kern = pl.kernel(
        body,
        out_type=jax.ShapeDtypeStruct((NC, NP, D), jnp.float32),
        mesh=_sc_mesh,
        scratch_types=[
            pltpu.VMEM((NCHUNK, CH), jnp.int32),
            pltpu.VMEM((CH, D), jnp.float32),
            pltpu.VMEM((ZR, D), jnp.float32),
            pltpu.VMEM_SHARED((NP, D), jnp.float32),
        ],
    )
    return kern(edges)


_BLK = 1000


def _tc_in_body(x_ref, wl_ref, wc_ref, bc_ref, y_ref, z_ref):
    xb = x_ref[...]
    y_ref[...] = jnp.dot(xb, wl_ref[...], preferred_element_type=jnp.float32)
    z_ref[...] = (jnp.dot(xb, wc_ref[...], preferred_element_type=jnp.float32)
                  + bc_ref[...])


def _tc_mid_body(a0_ref, a1_ref, d0_ref, d1_ref, z1_ref, wl_ref, wc_ref,
                 bc_ref, y2_ref, z2_ref):
    deg = jnp.clip(d0_ref[...][:, :1] + d1_ref[...][:, :1], 1.0, None)
    h = (a0_ref[...] + a1_ref[...]) / deg + z1_ref[...]
    h = jnp.where(h > 0, h, jnp.exp(jnp.minimum(h, 0.0)) - 1.0)
    y2_ref[...] = jnp.dot(h, wl_ref[...], preferred_element_type=jnp.float32)
    z2_ref[...] = (jnp.dot(h, wc_ref[...], preferred_element_type=jnp.float32)
                   + bc_ref[...])


def _tc_out_body(a0_ref, a1_ref, d0_ref, d1_ref, z2_ref, o_ref):
    deg = jnp.clip(d0_ref[...][:, :1] + d1_ref[...][:, :1], 1.0, None)
    o_ref[...] = (a0_ref[...] + a1_ref[...]) / deg + z2_ref[...]


def _row_spec(width):
    return pl.BlockSpec((_BLK, width), lambda i: (i, 0))


def _full_spec(shape):
    return pl.BlockSpec(shape, lambda i: (0,) * len(shape))


def kernel(x, edge_index, Wl1, bl1, Wr1, Wlin1, blin1, Wl2, bl2, Wr2,
           Wlin2, blin2):
    W1c = Wr1 + Wlin1
    b1c = (bl1 + blin1).reshape(1, D)
    pad = ((0, 0), (0, D - N_CLS))
    Wl2p = jnp.pad(Wl2, pad)
    W2c = jnp.pad(Wr2 + Wlin2, pad)
    b2c = jnp.pad((bl2 + blin2).reshape(1, N_CLS), ((0, 0), (0, D - N_CLS)))
    e = edge_index.reshape(2, NW, EPT)
    epad = jnp.broadcast_to(
        jnp.array([[0], [NP - 1]], dtype=jnp.int32).reshape(2, 1, 1),
        (2, NW, EPTP - EPT))
    edges = jnp.concatenate([e, epad], axis=2).reshape(2, NW, NCHUNK, CH)
    degp = _sc_degree(edges)

    grid = (N // _BLK,)
    y1, z1 = pl.pallas_call(
        _tc_in_body,
        grid=grid,
        in_specs=[_row_spec(D), _full_spec((D, D)), _full_spec((D, D)),
                  _full_spec((1, D))],
        out_specs=[_row_spec(D), _row_spec(D)],
        out_shape=[jax.ShapeDtypeStruct((N, D), jnp.float32)] * 2,
    )(x, Wl1, W1c, b1c)

    agg1 = _sc_segsum(y1, edges)

    y2, z2 = pl.pallas_call(
        _tc_mid_body,
        grid=grid,
        in_specs=[_row_spec(D), _row_spec(D), _row_spec(D), _row_spec(D),
                  _row_spec(D), _full_spec((D, D)), _full_spec((D, D)),
                  _full_spec((1, D))],
        out_specs=[_row_spec(D), _row_spec(D)],
        out_shape=[jax.ShapeDtypeStruct((N, D), jnp.float32)] * 2,
    )(agg1[0], agg1[1], degp[0], degp[1], z1, Wl2p, W2c, b2c)

    agg2 = _sc_segsum(y2, edges)

    out = pl.pallas_call(
        _tc_out_body,
        grid=grid,
        in_specs=[_row_spec(D), _row_spec(D), _row_spec(D), _row_spec(D),
                  _row_spec(D)],
        out_specs=_row_spec(D),
        out_shape=jax.ShapeDtypeStruct((N, D), jnp.float32),
    )(agg2[0], agg2[1], degp[0], degp[1], z2)

    return out[:, :N_CLS]
